# Initial kernel scaffold; baseline (speedup 1.0000x reference)
import functools

import jax
import jax.numpy as jnp
from jax import lax
from jax.experimental import pallas as pl
from jax.experimental.pallas import tpu as pltpu
from jax.experimental.pallas import tpu_sc as plsc

NX = 100000
NY = 100000
DX = 32
DY = 32
HIDDEN = 100
HIDDEN_PAD = 128
Q = 1024
BK = 2000
BIG_I32 = jnp.int32(2**30)


def _sc_gather(indexes, x_weight):
    info = plsc.get_sparse_core_info()
    nc, ns = info.num_cores, info.num_subcores
    nw = nc * ns
    b_per_w = Q // nw
    mesh = plsc.VectorSubcoreMesh(core_axis_name="c", subcore_axis_name="s")

    @functools.partial(
        pl.kernel,
        out_type=jax.ShapeDtypeStruct((Q, DX), jnp.float32),
        mesh=mesh,
        scratch_types=[
            pltpu.VMEM((b_per_w,), jnp.int32),
            pltpu.VMEM((b_per_w, DX), jnp.float32),
            pltpu.SemaphoreType.DMA,
        ],
    )
    def gather_kernel(idx_hbm, table_hbm, out_hbm, idx_v, rows_v, sem):
        wid = lax.axis_index("s") * nc + lax.axis_index("c")
        base = wid * b_per_w
        pltpu.sync_copy(idx_hbm.at[pl.ds(base, b_per_w)], idx_v)
        pltpu.async_copy(table_hbm.at[idx_v], rows_v, sem).wait()
        pltpu.sync_copy(rows_v, out_hbm.at[pl.ds(base, b_per_w)])

    return gather_kernel(indexes, x_weight)


def _tc_body(gx_ref, w1_ref, b1_ref, w2_ref, b2_ref, idx_ref, y_ref,
             out_ref, q_s, val_s, ind_s):
    k = pl.program_id(0)

    @pl.when(k == 0)
    def _init():
        h = jnp.dot(gx_ref[:, :], w1_ref[:, :],
                    preferred_element_type=jnp.float32) + b1_ref[:, :]
        h = jnp.maximum(h, 0.0)
        q_s[:, :] = jnp.dot(h, w2_ref[:, :],
                            preferred_element_type=jnp.float32) + b2_ref[:, :]
        val_s[:, :] = jnp.full((Q, 1), jnp.inf, dtype=jnp.float32)
        ind_s[:, :] = jnp.zeros((Q, 1), dtype=jnp.int32)

    yb = y_ref[:, :]
    tn = jnp.sum(yb * yb, axis=1)
    s = lax.dot_general(q_s[:, :], yb,
                        (((1,), (1,)), ((), ())),
                        preferred_element_type=jnp.float32)
    m = tn[None, :] - 2.0 * s
    bmin = jnp.min(m, axis=1, keepdims=True)
    iota = lax.broadcasted_iota(jnp.int32, m.shape, 1)
    bidx = jnp.min(jnp.where(m == bmin, iota, BIG_I32), axis=1, keepdims=True)
    better = bmin < val_s[:, :]
    val_s[:, :] = jnp.where(better, bmin, val_s[:, :])
    ind_s[:, :] = jnp.where(better, bidx + k * BK, ind_s[:, :])

    @pl.when(k == pl.num_programs(0) - 1)
    def _finish():
        mism = (ind_s[:, :] != idx_ref[:, :]).astype(jnp.float32)
        out_ref[0, 0] = jnp.sum(mism) / Q


def _tc_loss(gx, w1p, b1p, w2p, b2p, idx2d, y_weight, interpret=False):
    grid = NY // BK
    return pl.pallas_call(
        _tc_body,
        grid=(grid,),
        in_specs=[
            pl.BlockSpec((Q, DX), lambda k: (0, 0)),
            pl.BlockSpec((DX, HIDDEN_PAD), lambda k: (0, 0)),
            pl.BlockSpec((1, HIDDEN_PAD), lambda k: (0, 0)),
            pl.BlockSpec((HIDDEN_PAD, DY), lambda k: (0, 0)),
            pl.BlockSpec((1, DY), lambda k: (0, 0)),
            pl.BlockSpec((Q, 1), lambda k: (0, 0)),
            pl.BlockSpec((BK, DY), lambda k: (k, 0)),
        ],
        out_specs=pl.BlockSpec(memory_space=pltpu.SMEM),
        out_shape=jax.ShapeDtypeStruct((1, 1), jnp.float32),
        scratch_shapes=[
            pltpu.VMEM((Q, DY), jnp.float32),
            pltpu.VMEM((Q, 1), jnp.float32),
            pltpu.VMEM((Q, 1), jnp.int32),
        ],
        compiler_params=pltpu.CompilerParams(
            dimension_semantics=("arbitrary",),
        ),
        interpret=interpret,
    )(gx, w1p, b1p, w2p, b2p, idx2d, y_weight)


def kernel(x_weight, y_weight, W1, b1, W2, b2, indexes):
    gx = _sc_gather(indexes, x_weight)
    w1p = jnp.zeros((DX, HIDDEN_PAD), jnp.float32).at[:, :HIDDEN].set(W1)
    b1p = jnp.zeros((1, HIDDEN_PAD), jnp.float32).at[0, :HIDDEN].set(b1)
    w2p = jnp.zeros((HIDDEN_PAD, DY), jnp.float32).at[:HIDDEN, :].set(W2)
    b2p = b2.reshape(1, DY)
    idx2d = indexes.reshape(Q, 1)
    out = _tc_loss(gx, w1p, b1p, w2p, b2p, idx2d, y_weight)
    return out[0, 0]


# speedup vs baseline: 3.5921x; 3.5921x over previous
import functools

import jax
import jax.numpy as jnp
from jax import lax
from jax.experimental import pallas as pl
from jax.experimental.pallas import tpu as pltpu
from jax.experimental.pallas import tpu_sc as plsc

NX = 100000
NY = 100000
DX = 32
DY = 32
HIDDEN = 100
HIDDEN_PAD = 128
Q = 1024
BK = 1000
BIG_I32 = 2**30


def _sc_gather(indexes, x_weight):
    info = plsc.get_sparse_core_info()
    nc, ns = info.num_cores, info.num_subcores
    nw = nc * ns
    b_per_w = Q // nw
    mesh = plsc.VectorSubcoreMesh(core_axis_name="c", subcore_axis_name="s")

    @functools.partial(
        pl.kernel,
        out_type=jax.ShapeDtypeStruct((Q, DX), jnp.float32),
        mesh=mesh,
        scratch_types=[
            pltpu.VMEM((b_per_w,), jnp.int32),
            pltpu.VMEM((b_per_w, DX), jnp.float32),
            pltpu.SemaphoreType.DMA,
        ],
        compiler_params=pltpu.CompilerParams(use_tc_tiling_on_sc=False),
    )
    def gather_kernel(idx_hbm, table_hbm, out_hbm, idx_v, rows_v, sem):
        wid = lax.axis_index("s") * nc + lax.axis_index("c")
        base = wid * b_per_w
        pltpu.sync_copy(idx_hbm.at[pl.ds(base, b_per_w)], idx_v)
        pltpu.async_copy(table_hbm.at[idx_v], rows_v, sem).wait()
        pltpu.sync_copy(rows_v, out_hbm.at[pl.ds(base, b_per_w)])

    return gather_kernel(indexes, x_weight)


def _tc_body(gxt_ref, w1t_ref, b1c_ref, w2t_ref, b2c_ref, idx_ref, y_ref,
             out_ref, qt_s, val_s, ind_s):
    k = pl.program_id(0)

    @pl.when(k == 0)
    def _init():
        h = jnp.dot(w1t_ref[:, :], gxt_ref[:, :],
                    preferred_element_type=jnp.float32) + b1c_ref[:, :]
        h = jnp.maximum(h, 0.0)
        qt_s[:, :] = jnp.dot(w2t_ref[:, :], h,
                             preferred_element_type=jnp.float32) + b2c_ref[:, :]
        val_s[:, :] = jnp.full((1, Q), jnp.inf, dtype=jnp.float32)
        ind_s[:, :] = jnp.zeros((1, Q), dtype=jnp.int32)

    yb = y_ref[:, :]
    tn = jnp.sum(yb * yb, axis=1, keepdims=True)
    s = jnp.dot(yb, qt_s[:, :],
                preferred_element_type=jnp.float32)
    m = tn - 2.0 * s
    bmin = jnp.min(m, axis=0, keepdims=True)
    iota = lax.broadcasted_iota(jnp.int32, m.shape, 0)
    bidx = jnp.min(jnp.where(m == bmin, iota, BIG_I32),
                   axis=0, keepdims=True)
    better = bmin < val_s[:, :]
    val_s[:, :] = jnp.where(better, bmin, val_s[:, :])
    ind_s[:, :] = jnp.where(better, bidx + k * BK, ind_s[:, :])

    @pl.when(k == pl.num_programs(0) - 1)
    def _finish():
        mism = (ind_s[:, :] != idx_ref[:, :]).astype(jnp.float32)
        out_ref[0, 0] = jnp.sum(mism) / Q


def _tc_loss(gxt, w1t, b1c, w2t, b2c, idxrow, y_weight, interpret=False):
    grid = NY // BK
    return pl.pallas_call(
        _tc_body,
        grid=(grid,),
        in_specs=[
            pl.BlockSpec((DX, Q), lambda k: (0, 0)),
            pl.BlockSpec((HIDDEN_PAD, DX), lambda k: (0, 0)),
            pl.BlockSpec((HIDDEN_PAD, 1), lambda k: (0, 0)),
            pl.BlockSpec((DY, HIDDEN_PAD), lambda k: (0, 0)),
            pl.BlockSpec((DY, 1), lambda k: (0, 0)),
            pl.BlockSpec((1, Q), lambda k: (0, 0)),
            pl.BlockSpec((BK, DY), lambda k: (k, 0)),
        ],
        out_specs=pl.BlockSpec(memory_space=pltpu.SMEM),
        out_shape=jax.ShapeDtypeStruct((1, 1), jnp.float32),
        scratch_shapes=[
            pltpu.VMEM((DY, Q), jnp.float32),
            pltpu.VMEM((1, Q), jnp.float32),
            pltpu.VMEM((1, Q), jnp.int32),
        ],
        compiler_params=pltpu.CompilerParams(
            dimension_semantics=("arbitrary",),
        ),
        interpret=interpret,
    )(gxt, w1t, b1c, w2t, b2c, idxrow, y_weight)


def kernel(x_weight, y_weight, W1, b1, W2, b2, indexes):
    gx = _sc_gather(indexes, x_weight)
    gxt = gx.T
    w1t = jnp.zeros((HIDDEN_PAD, DX), jnp.float32).at[:HIDDEN, :].set(W1.T)
    b1c = jnp.zeros((HIDDEN_PAD, 1), jnp.float32).at[:HIDDEN, 0].set(b1)
    w2t = jnp.zeros((DY, HIDDEN_PAD), jnp.float32).at[:, :HIDDEN].set(W2.T)
    b2c = b2.reshape(DY, 1)
    idxrow = indexes.reshape(1, Q)
    out = _tc_loss(gxt, w1t, b1c, w2t, b2c, idxrow, y_weight)
    return out[0, 0]


# speedup vs baseline: 3.9690x; 1.1049x over previous
import functools

import jax
import jax.numpy as jnp
from jax import lax
from jax.experimental import pallas as pl
from jax.experimental.pallas import tpu as pltpu
from jax.experimental.pallas import tpu_sc as plsc

NX = 100000
NY = 100000
DX = 32
DY = 32
HIDDEN = 100
HIDDEN_PAD = 128
Q = 1024
BK = 2000
BIG_I32 = 2**30


def _sc_gather(indexes, x_weight):
    info = plsc.get_sparse_core_info()
    nc, ns = info.num_cores, info.num_subcores
    nw = nc * ns
    b_per_w = Q // nw
    mesh = plsc.VectorSubcoreMesh(core_axis_name="c", subcore_axis_name="s")

    @functools.partial(
        pl.kernel,
        out_type=jax.ShapeDtypeStruct((Q, DX), jnp.float32),
        mesh=mesh,
        scratch_types=[
            pltpu.VMEM((b_per_w,), jnp.int32),
            pltpu.VMEM((b_per_w, DX), jnp.float32),
            pltpu.SemaphoreType.DMA,
        ],
        compiler_params=pltpu.CompilerParams(use_tc_tiling_on_sc=False),
    )
    def gather_kernel(idx_hbm, table_hbm, out_hbm, idx_v, rows_v, sem):
        wid = lax.axis_index("s") * nc + lax.axis_index("c")
        base = wid * b_per_w
        pltpu.sync_copy(idx_hbm.at[pl.ds(base, b_per_w)], idx_v)
        pltpu.async_copy(table_hbm.at[idx_v], rows_v, sem).wait()
        pltpu.sync_copy(rows_v, out_hbm.at[pl.ds(base, b_per_w)])

    return gather_kernel(indexes, x_weight)


def _tc_body(gxt_ref, w1t_ref, b1c_ref, w2t_ref, b2c_ref, idx_ref, y_ref,
             out_ref, qt_s, val_s, ind_s):
    k = pl.program_id(0)

    @pl.when(k == 0)
    def _init():
        h = jnp.dot(w1t_ref[:, :], gxt_ref[:, :],
                    preferred_element_type=jnp.float32) + b1c_ref[:, :]
        h = jnp.maximum(h, 0.0)
        qt_s[0:DY, :] = jnp.dot(w2t_ref[:, :], h,
                                preferred_element_type=jnp.float32) + b2c_ref[:, :]
        qt_s[DY:DY + 8, :] = jnp.ones((8, Q), dtype=jnp.float32)
        val_s[:, :] = jnp.full((1, Q), jnp.inf, dtype=jnp.float32)
        ind_s[:, :] = jnp.zeros((1, Q), dtype=jnp.int32)

    yb = y_ref[:, :]
    tn = jnp.sum(yb * yb, axis=1, keepdims=True)
    yaug = jnp.concatenate(
        [yb * -2.0, tn, jnp.zeros((BK, 7), dtype=jnp.float32)], axis=1)
    m = jnp.dot(yaug, qt_s[:, :],
                preferred_element_type=jnp.float32)
    bmin = jnp.min(m, axis=0, keepdims=True)
    iota = lax.broadcasted_iota(jnp.int32, m.shape, 0)
    bidx = jnp.min(jnp.where(m == bmin, iota, BIG_I32),
                   axis=0, keepdims=True)
    better = bmin < val_s[:, :]
    val_s[:, :] = jnp.where(better, bmin, val_s[:, :])
    ind_s[:, :] = jnp.where(better, bidx + k * BK, ind_s[:, :])

    @pl.when(k == pl.num_programs(0) - 1)
    def _finish():
        mism = (ind_s[:, :] != idx_ref[:, :]).astype(jnp.float32)
        out_ref[0, 0] = jnp.sum(mism) / Q


def _tc_loss(gxt, w1t, b1c, w2t, b2c, idxrow, y_weight, interpret=False):
    grid = NY // BK
    return pl.pallas_call(
        _tc_body,
        grid=(grid,),
        in_specs=[
            pl.BlockSpec((DX, Q), lambda k: (0, 0)),
            pl.BlockSpec((HIDDEN_PAD, DX), lambda k: (0, 0)),
            pl.BlockSpec((HIDDEN_PAD, 1), lambda k: (0, 0)),
            pl.BlockSpec((DY, HIDDEN_PAD), lambda k: (0, 0)),
            pl.BlockSpec((DY, 1), lambda k: (0, 0)),
            pl.BlockSpec((1, Q), lambda k: (0, 0)),
            pl.BlockSpec((BK, DY), lambda k: (k, 0)),
        ],
        out_specs=pl.BlockSpec(memory_space=pltpu.SMEM),
        out_shape=jax.ShapeDtypeStruct((1, 1), jnp.float32),
        scratch_shapes=[
            pltpu.VMEM((DY + 8, Q), jnp.float32),
            pltpu.VMEM((1, Q), jnp.float32),
            pltpu.VMEM((1, Q), jnp.int32),
        ],
        compiler_params=pltpu.CompilerParams(
            dimension_semantics=("arbitrary",),
        ),
        interpret=interpret,
    )(gxt, w1t, b1c, w2t, b2c, idxrow, y_weight)


def kernel(x_weight, y_weight, W1, b1, W2, b2, indexes):
    gx = _sc_gather(indexes, x_weight)
    gxt = gx.T
    w1t = jnp.zeros((HIDDEN_PAD, DX), jnp.float32).at[:HIDDEN, :].set(W1.T)
    b1c = jnp.zeros((HIDDEN_PAD, 1), jnp.float32).at[:HIDDEN, 0].set(b1)
    w2t = jnp.zeros((DY, HIDDEN_PAD), jnp.float32).at[:, :HIDDEN].set(W2.T)
    b2c = b2.reshape(DY, 1)
    idxrow = indexes.reshape(1, Q)
    out = _tc_loss(gxt, w1t, b1c, w2t, b2c, idxrow, y_weight)
    return out[0, 0]
